# baseline (speedup 1.0000x reference)
import functools

import jax
import jax.numpy as jnp
from jax import lax
from jax.experimental import pallas as pl
from jax.experimental.pallas import tpu as pltpu
from jax.experimental.pallas import tpu_sc as plsc

HIDDEN = 1024
D_FF = 1024
N_EXP = 8
N_LOGITS = 16
TOKENS = 2048
TT = 256


def _router_body(x_ref, wr_ref, br_ref, cb_ref, gates_ref, zw_ref):
    x = x_ref[...]
    logits = jax.lax.dot_general(
        x, wr_ref[...], (((1,), (0,)), ((), ())),
        preferred_element_type=jnp.float32) + br_ref[...]
    scores = jax.nn.sigmoid(logits)
    biased = scores + cb_ref[...]
    col = jax.lax.broadcasted_iota(jnp.int32, biased.shape, 1)
    m1 = jnp.max(biased, axis=1, keepdims=True)
    i1 = jnp.min(jnp.where(biased == m1, col, N_LOGITS), axis=1, keepdims=True)
    sel1 = col == i1
    w1 = jnp.sum(jnp.where(sel1, scores, 0.0), axis=1, keepdims=True)
    b2 = jnp.where(sel1, -jnp.inf, biased)
    m2 = jnp.max(b2, axis=1, keepdims=True)
    i2 = jnp.min(jnp.where(b2 == m2, col, N_LOGITS), axis=1, keepdims=True)
    sel2 = col == i2
    w2 = jnp.sum(jnp.where(sel2, scores, 0.0), axis=1, keepdims=True)

    ecol = jax.lax.broadcasted_iota(jnp.int32, (TT, N_EXP), 1)
    g1 = jnp.where((i1 == ecol) & (i1 < N_EXP), w1, 0.0)
    g2 = jnp.where((i2 == ecol) & (i2 < N_EXP), w2, 0.0)
    gates_ref[...] = g1 + g2
    zw_ref[...] = (jnp.where(i1 >= N_EXP, w1, 0.0)
                   + jnp.where(i2 >= N_EXP, w2, 0.0))


def _moe_body(gates_ref, x_ref, w1_ref, w2_ref, out_ref):
    e = pl.program_id(1)
    x = x_ref[...]
    h = jnp.dot(x.astype(jnp.bfloat16), w1_ref[e],
                preferred_element_type=jnp.float32)
    gate = h[:, :D_FF]
    up = h[:, D_FF:]
    act = gate * jax.nn.sigmoid(gate) * up
    o = jnp.dot(act.astype(jnp.bfloat16), w2_ref[e],
                preferred_element_type=jnp.float32)
    ecol = jax.lax.broadcasted_iota(jnp.int32, (TT, N_EXP), 1)
    g = jnp.sum(jnp.where(ecol == e, gates_ref[...], 0.0), axis=1,
                keepdims=True)
    contrib = g * o

    @pl.when(e == 0)
    def _():
        out_ref[...] = contrib

    @pl.when(e > 0)
    def _():
        out_ref[...] += contrib


def _add_body(a_ref, b_ref, o_ref):
    o_ref[...] = a_ref[...] + b_ref[...]


def _i16(c):
    return jnp.full((16,), c, jnp.int32)


def _dyn_gather16(y, idx):
    return jax.lax.gather(
        y, idx[:, None],
        jax.lax.GatherDimensionNumbers(
            offset_dims=(), collapsed_slice_dims=(0,), start_index_map=(0,)),
        (1,), mode=jax.lax.GatherScatterMode.PROMISE_IN_BOUNDS)


def _zwx_body(zw_hbm, x_hbm, zwx_hbm, zw_loc, x_loc, lanes_f_unused):
    cid = lax.axis_index("c")
    sid = lax.axis_index("s")
    wid = sid * 2 + cid
    tpw = TOKENS // 32
    t0 = wid * tpw
    pltpu.sync_copy(zw_hbm.at[pl.ds(t0, tpw)], zw_loc)
    pltpu.sync_copy(x_hbm.at[pl.ds(t0, tpw)], x_loc)

    def row(i, _):
        zsplat = _dyn_gather16(zw_loc[...], _i16(0) + i)

        def vec(j, _):
            sl = pl.ds(j * 16, 16)
            x_loc[i, sl] = x_loc[i, sl] * zsplat
            return 0

        return lax.fori_loop(0, HIDDEN // 16, vec, 0)

    lax.fori_loop(0, tpw, row, 0)
    pltpu.sync_copy(x_loc, zwx_hbm.at[pl.ds(t0, tpw)])


_sc_mesh = plsc.VectorSubcoreMesh(core_axis_name="c", subcore_axis_name="s")

_zwx = pl.kernel(
    _zwx_body, mesh=_sc_mesh,
    out_type=[jax.ShapeDtypeStruct((TOKENS, HIDDEN), jnp.float32)],
    scratch_types=[
        pltpu.VMEM((TOKENS // 32,), jnp.float32),
        pltpu.VMEM((TOKENS // 32, HIDDEN), jnp.float32),
        pltpu.VMEM((16,), jnp.float32),
    ],
)


def kernel(hidden_states, Wr, br, correction_bias, W1, W2):
    T = hidden_states.shape[0]
    nt = T // TT
    br2 = br.reshape(1, N_LOGITS)
    cb2 = correction_bias.reshape(1, N_LOGITS)

    gates, zw = pl.pallas_call(
        _router_body,
        grid=(nt,),
        in_specs=[
            pl.BlockSpec((TT, HIDDEN), lambda t: (t, 0)),
            pl.BlockSpec((HIDDEN, N_LOGITS), lambda t: (0, 0)),
            pl.BlockSpec((1, N_LOGITS), lambda t: (0, 0)),
            pl.BlockSpec((1, N_LOGITS), lambda t: (0, 0)),
        ],
        out_specs=[
            pl.BlockSpec((TT, N_EXP), lambda t: (t, 0)),
            pl.BlockSpec((TT, 1), lambda t: (t, 0)),
        ],
        out_shape=[
            jax.ShapeDtypeStruct((T, N_EXP), jnp.float32),
            jax.ShapeDtypeStruct((T, 1), jnp.float32),
        ],
    )(hidden_states, Wr, br2, cb2)

    out = pl.pallas_call(
        _moe_body,
        grid=(nt, N_EXP),
        in_specs=[
            pl.BlockSpec((TT, N_EXP), lambda t, e: (t, 0)),
            pl.BlockSpec((TT, HIDDEN), lambda t, e: (t, 0)),
            pl.BlockSpec((N_EXP, HIDDEN, 2 * D_FF), lambda t, e: (0, 0, 0)),
            pl.BlockSpec((N_EXP, D_FF, HIDDEN), lambda t, e: (0, 0, 0)),
        ],
        out_specs=pl.BlockSpec((TT, HIDDEN), lambda t, e: (t, 0)),
        out_shape=jax.ShapeDtypeStruct((T, HIDDEN), jnp.float32),
    )(gates, hidden_states, W1.astype(jnp.bfloat16),
      W2.astype(jnp.bfloat16))

    (zwx,) = _zwx(zw.reshape(T), hidden_states)

    out = pl.pallas_call(
        _add_body,
        grid=(nt,),
        in_specs=[
            pl.BlockSpec((TT, HIDDEN), lambda t: (t, 0)),
            pl.BlockSpec((TT, HIDDEN), lambda t: (t, 0)),
        ],
        out_specs=pl.BlockSpec((TT, HIDDEN), lambda t: (t, 0)),
        out_shape=jax.ShapeDtypeStruct((T, HIDDEN), jnp.float32),
    )(out, zwx)
    return out


# speedup vs baseline: 1.1437x; 1.1437x over previous
import functools

import jax
import jax.numpy as jnp
from jax import lax
from jax.experimental import pallas as pl
from jax.experimental.pallas import tpu as pltpu
from jax.experimental.pallas import tpu_sc as plsc

HIDDEN = 1024
D_FF = 1024
N_EXP = 8
N_LOGITS = 16
TOKENS = 2048
TT = 256


def _router_body(x_ref, wr_ref, br_ref, cb_ref, gates_ref, zw_ref):
    x = x_ref[...]
    logits = jax.lax.dot_general(
        x, wr_ref[...], (((1,), (0,)), ((), ())),
        preferred_element_type=jnp.float32) + br_ref[...]
    scores = jax.nn.sigmoid(logits)
    biased = scores + cb_ref[...]
    col = jax.lax.broadcasted_iota(jnp.int32, biased.shape, 1)
    m1 = jnp.max(biased, axis=1, keepdims=True)
    i1 = jnp.min(jnp.where(biased == m1, col, N_LOGITS), axis=1, keepdims=True)
    sel1 = col == i1
    w1 = jnp.sum(jnp.where(sel1, scores, 0.0), axis=1, keepdims=True)
    b2 = jnp.where(sel1, -jnp.inf, biased)
    m2 = jnp.max(b2, axis=1, keepdims=True)
    i2 = jnp.min(jnp.where(b2 == m2, col, N_LOGITS), axis=1, keepdims=True)
    sel2 = col == i2
    w2 = jnp.sum(jnp.where(sel2, scores, 0.0), axis=1, keepdims=True)

    ecol = jax.lax.broadcasted_iota(jnp.int32, (TT, N_EXP), 1)
    g1 = jnp.where((i1 == ecol) & (i1 < N_EXP), w1, 0.0)
    g2 = jnp.where((i2 == ecol) & (i2 < N_EXP), w2, 0.0)
    gates_ref[...] = g1 + g2
    zw_ref[...] = (jnp.where(i1 >= N_EXP, w1, 0.0)
                   + jnp.where(i2 >= N_EXP, w2, 0.0))


def _moe_body(gates_ref, zw_ref, x_ref, w1_ref, w2_ref, out_ref):
    e = pl.program_id(1)
    x = x_ref[...]
    h = jnp.dot(x.astype(jnp.bfloat16), w1_ref[e],
                preferred_element_type=jnp.float32)
    gate = h[:, :D_FF]
    up = h[:, D_FF:]
    act = gate * jax.nn.sigmoid(gate) * up
    o = jnp.dot(act.astype(jnp.bfloat16), w2_ref[e],
                preferred_element_type=jnp.float32)
    ecol = jax.lax.broadcasted_iota(jnp.int32, (TT, N_EXP), 1)
    g = jnp.sum(jnp.where(ecol == e, gates_ref[...], 0.0), axis=1,
                keepdims=True)
    contrib = g * o

    @pl.when(e == 0)
    def _():
        out_ref[...] = zw_ref[...] * x + contrib

    @pl.when(e > 0)
    def _():
        out_ref[...] += contrib


def _add_body(a_ref, b_ref, o_ref):
    o_ref[...] = a_ref[...] + b_ref[...]


def _i16(c):
    return jnp.full((16,), c, jnp.int32)


def _dyn_gather16(y, idx):
    return jax.lax.gather(
        y, idx[:, None],
        jax.lax.GatherDimensionNumbers(
            offset_dims=(), collapsed_slice_dims=(0,), start_index_map=(0,)),
        (1,), mode=jax.lax.GatherScatterMode.PROMISE_IN_BOUNDS)


def _zwx_body(zw_hbm, x_hbm, zwx_hbm, zw_loc, x_loc, lanes_f_unused):
    cid = lax.axis_index("c")
    sid = lax.axis_index("s")
    wid = sid * 2 + cid
    tpw = TOKENS // 32
    t0 = wid * tpw
    pltpu.sync_copy(zw_hbm.at[pl.ds(t0, tpw)], zw_loc)
    pltpu.sync_copy(x_hbm.at[pl.ds(t0, tpw)], x_loc)

    def row(i, _):
        zsplat = _dyn_gather16(zw_loc[...], _i16(0) + i)

        def vec(j, _):
            sl = pl.ds(j * 16, 16)
            x_loc[i, sl] = x_loc[i, sl] * zsplat
            return 0

        return lax.fori_loop(0, HIDDEN // 16, vec, 0)

    lax.fori_loop(0, tpw, row, 0)
    pltpu.sync_copy(x_loc, zwx_hbm.at[pl.ds(t0, tpw)])


_sc_mesh = plsc.VectorSubcoreMesh(core_axis_name="c", subcore_axis_name="s")

_zwx = pl.kernel(
    _zwx_body, mesh=_sc_mesh,
    out_type=[jax.ShapeDtypeStruct((TOKENS, HIDDEN), jnp.float32)],
    scratch_types=[
        pltpu.VMEM((TOKENS // 32,), jnp.float32),
        pltpu.VMEM((TOKENS // 32, HIDDEN), jnp.float32),
        pltpu.VMEM((16,), jnp.float32),
    ],
)


def kernel(hidden_states, Wr, br, correction_bias, W1, W2):
    T = hidden_states.shape[0]
    nt = T // TT
    br2 = br.reshape(1, N_LOGITS)
    cb2 = correction_bias.reshape(1, N_LOGITS)

    gates, zw = pl.pallas_call(
        _router_body,
        grid=(nt,),
        in_specs=[
            pl.BlockSpec((TT, HIDDEN), lambda t: (t, 0)),
            pl.BlockSpec((HIDDEN, N_LOGITS), lambda t: (0, 0)),
            pl.BlockSpec((1, N_LOGITS), lambda t: (0, 0)),
            pl.BlockSpec((1, N_LOGITS), lambda t: (0, 0)),
        ],
        out_specs=[
            pl.BlockSpec((TT, N_EXP), lambda t: (t, 0)),
            pl.BlockSpec((TT, 1), lambda t: (t, 0)),
        ],
        out_shape=[
            jax.ShapeDtypeStruct((T, N_EXP), jnp.float32),
            jax.ShapeDtypeStruct((T, 1), jnp.float32),
        ],
    )(hidden_states, Wr, br2, cb2)

    out = pl.pallas_call(
        _moe_body,
        grid=(nt, N_EXP),
        in_specs=[
            pl.BlockSpec((TT, N_EXP), lambda t, e: (t, 0)),
            pl.BlockSpec((TT, 1), lambda t, e: (t, 0)),
            pl.BlockSpec((TT, HIDDEN), lambda t, e: (t, 0)),
            pl.BlockSpec((N_EXP, HIDDEN, 2 * D_FF), lambda t, e: (0, 0, 0)),
            pl.BlockSpec((N_EXP, D_FF, HIDDEN), lambda t, e: (0, 0, 0)),
        ],
        out_specs=pl.BlockSpec((TT, HIDDEN), lambda t, e: (t, 0)),
        out_shape=jax.ShapeDtypeStruct((T, HIDDEN), jnp.float32),
    )(gates, zw, hidden_states, W1.astype(jnp.bfloat16),
      W2.astype(jnp.bfloat16))
    return out
